# Initial kernel scaffold; baseline (speedup 1.0000x reference)
import jax
import jax.numpy as jnp
from jax.experimental import pallas as pl


def kernel(sample, logits, gumbel, net_logits, gumbel_noise, states, position):
    raise NotImplementedError("write your pallas kernel here")


# speedup vs baseline: 3.5985x; 3.5985x over previous
import functools

import jax
import jax.numpy as jnp
from jax import lax
from jax.experimental import pallas as pl
from jax.experimental.pallas import tpu as pltpu
from jax.experimental.pallas import tpu_sc as plsc

N, H, L, D = 16384, 4, 200, 256
NH = N * H
RS, CS = 512, 128
NW = 32


def _sort_body(g_ref, lg_ref, gs_ref, ls_ref, ridx4_ref, hval_ref, srow_ref,
               K_ref, I_ref, Lg_ref):
    rowi = lax.broadcasted_iota(jnp.int32, (RS, CS), 0)
    coli = lax.broadcasted_iota(jnp.int32, (RS, CS), 1)
    fio = rowi * CS + coli

    u = lax.bitcast_convert_type(g_ref[...], jnp.int32)
    K_ref[...] = u ^ ((u >> 31) & 0x7FFFFFFF)
    I_ref[...] = fio
    Lg_ref[...] = lg_ref[...]

    def one_pass(j, k):
        K = K_ref[...]
        I = I_ref[...]
        Lg = Lg_ref[...]
        d = 1 << j
        if j < 7:
            axis, s, size = 1, d, CS
            bsel = (coli & d) != 0
        else:
            axis, s, size = 0, d >> 7, RS
            bsel = (rowi & (d >> 7)) != 0

        def pt(x):
            return jnp.where(bsel, pltpu.roll(x, s, axis),
                             pltpu.roll(x, size - s, axis))

        Kp, Ip, Lp = pt(K), pt(I), pt(Lg)
        Lt = (K > Kp) | ((K == Kp) & (I < Ip))
        dirb = ((fio >> k) & 1) != 0
        keep = Lt ^ dirb ^ bsel
        K_ref[...] = jnp.where(keep, K, Kp)
        I_ref[...] = jnp.where(keep, I, Ip)
        Lg_ref[...] = jnp.where(keep, Lg, Lp)

    def merge_body(k, _):
        for j in range(15, -1, -1):
            @pl.when(j < k)
            def _():
                one_pass(j, k)
        return 0

    lax.fori_loop(1, 17, merge_body, 0)

    K = K_ref[...]
    I = I_ref[...]
    gs_ref[...] = lax.bitcast_convert_type(K ^ ((K >> 31) & 0x7FFFFFFF),
                                           jnp.float32)
    ls_ref[...] = Lg_ref[...]
    ridx4_ref[...] = I & jnp.int32(-4)
    hval_ref[...] = I & 3
    srow_ref[...] = (I >> 2)[:128]


_tc_sort = pl.pallas_call(
    _sort_body,
    out_shape=(
        jax.ShapeDtypeStruct((RS, CS), jnp.float32),
        jax.ShapeDtypeStruct((RS, CS), jnp.float32),
        jax.ShapeDtypeStruct((RS, CS), jnp.int32),
        jax.ShapeDtypeStruct((RS, CS), jnp.int32),
        jax.ShapeDtypeStruct((128, CS), jnp.int32),
    ),
    scratch_shapes=[
        pltpu.VMEM((RS, CS), jnp.int32),
        pltpu.VMEM((RS, CS), jnp.int32),
        pltpu.VMEM((RS, CS), jnp.float32),
    ],
)


_S_PER_W = N // NW
_S_CHUNK = 128
_Q_PER_W = NH // NW
_Q_CHUNK = 128


@functools.cache
def _sc_kernels():
    mesh = plsc.VectorSubcoreMesh(core_axis_name="c", subcore_axis_name="s",
                                  num_cores=2, num_subcores=16)

    @functools.partial(
        pl.kernel,
        out_type=jax.ShapeDtypeStruct((N, D), jnp.float32),
        mesh=mesh,
        scratch_types=[
            pltpu.VMEM((_S_CHUNK,), jnp.int32),
            pltpu.VMEM((_S_CHUNK, D), jnp.float32),
            pltpu.SemaphoreType.DMA,
        ],
    )
    def sc_states(states_hbm, sidx_hbm, out_hbm, idx_v, rows_v, sem):
        wid = lax.axis_index("s") * 2 + lax.axis_index("c")
        base = wid * _S_PER_W
        for c in range(_S_PER_W // _S_CHUNK):
            b = base + c * _S_CHUNK
            pltpu.sync_copy(sidx_hbm.at[pl.ds(b, _S_CHUNK)], idx_v)
            pltpu.async_copy(states_hbm.at[idx_v], rows_v, sem).wait()
            pltpu.sync_copy(rows_v, out_hbm.at[pl.ds(b, _S_CHUNK)])

    @functools.partial(
        pl.kernel,
        out_type=jax.ShapeDtypeStruct((NH, L), jnp.int32),
        mesh=mesh,
        scratch_types=[
            pltpu.VMEM((_Q_CHUNK,), jnp.int32),
            pltpu.VMEM((_Q_CHUNK,), jnp.int32),
            pltpu.VMEM((16,), jnp.int32),
            pltpu.VMEM((_Q_CHUNK, L), jnp.int32),
            pltpu.SemaphoreType.DMA,
        ],
        compiler_params=pltpu.CompilerParams(use_tc_tiling_on_sc=False,
                                             needs_layout_passes=False),
    )
    def sc_sample(src_hbm, ridx4_hbm, hval_hbm, pos_hbm, out_hbm,
                  idx_v, hv_v, pos_v, row_v, sem):
        wid = lax.axis_index("s") * 2 + lax.axis_index("c")
        base = wid * _Q_PER_W
        pltpu.sync_copy(pos_hbm, pos_v)
        colv = pos_v[...]
        for c in range(_Q_PER_W // _Q_CHUNK):
            b = base + c * _Q_CHUNK
            pltpu.sync_copy(ridx4_hbm.at[pl.ds(b, _Q_CHUNK)], idx_v)
            pltpu.sync_copy(hval_hbm.at[pl.ds(b, _Q_CHUNK)], hv_v)
            pltpu.async_copy(src_hbm.at[idx_v], row_v, sem).wait()
            for t in range(_Q_CHUNK // 16):
                rvec = lax.iota(jnp.int32, 16) + t * 16
                hvec = hv_v[pl.ds(t * 16, 16)]
                plsc.store_scatter(row_v, (rvec, colv), hvec)
            pltpu.sync_copy(row_v, out_hbm.at[pl.ds(b, _Q_CHUNK)])

    return sc_states, sc_sample


def kernel(sample, logits, gumbel, net_logits, gumbel_noise, states, position):
    logitnew = logits[:, 0:1] + net_logits
    graw = logitnew + gumbel_noise
    Z = jnp.nanmax(graw, axis=1, keepdims=True)
    gnew = jnp.nan_to_num(
        -jnp.log(jnp.exp(-gumbel[:, 0:1]) - jnp.exp(-Z) + jnp.exp(-graw)),
        nan=-jnp.inf,
    )

    gs, ls, ridx4_p, hval_p, srow_p = _tc_sort(
        gnew.reshape(RS, CS), logitnew.reshape(RS, CS))

    out_gumbel = gs.reshape(H, N).T
    out_logits = ls.reshape(H, N).T
    ridx4_q = ridx4_p.reshape(H, N).T.reshape(-1)
    hval_q = hval_p.reshape(H, N).T.reshape(-1)
    sidx = srow_p.reshape(-1)

    sc_states, sc_sample = _sc_kernels()
    out_states = sc_states(states, sidx)

    posv = jnp.full((16,), position, dtype=jnp.int32)
    out_sample = sc_sample(
        sample.reshape(NH, L), ridx4_q, hval_q, posv).reshape(N, H, L)

    return (out_sample, out_logits, out_gumbel, out_states)


# speedup vs baseline: 3.7242x; 1.0349x over previous
import functools

import jax
import jax.numpy as jnp
from jax import lax
from jax.experimental import pallas as pl
from jax.experimental.pallas import tpu as pltpu
from jax.experimental.pallas import tpu_sc as plsc

N, H, L, D = 16384, 4, 200, 256
NH = N * H
RS, CS = 512, 128
NW = 32

S_PW, S_CH = N // NW, 64
Q_PW, Q_CH = NH // NW, 128


def _sort_body(g_ref, gs_ref, iq_ref, K_ref, I_ref):
    rowi = lax.broadcasted_iota(jnp.int32, (RS, CS), 0)
    coli = lax.broadcasted_iota(jnp.int32, (RS, CS), 1)
    fio = rowi * CS + coli

    u = lax.bitcast_convert_type(g_ref[...], jnp.int32)
    K_ref[...] = u ^ ((u >> 31) & 0x7FFFFFFF)
    I_ref[...] = fio

    def one_pass(j, k):
        K = K_ref[...]
        I = I_ref[...]
        d = 1 << j
        if j < 7:
            axis, s, size = 1, d, CS
            bsel = (coli & d) != 0
        else:
            axis, s, size = 0, d >> 7, RS
            bsel = (rowi & (d >> 7)) != 0

        def pt(x):
            return jnp.where(bsel, pltpu.roll(x, s, axis),
                             pltpu.roll(x, size - s, axis))

        Kp, Ip = pt(K), pt(I)
        Lt = (K > Kp) | ((K == Kp) & (I < Ip))
        dirb = ((fio >> k) & 1) != 0
        keep = Lt ^ dirb ^ bsel
        K_ref[...] = jnp.where(keep, K, Kp)
        I_ref[...] = jnp.where(keep, I, Ip)

    def merge_body(k, _):
        for j in range(15, -1, -1):
            @pl.when(j < k)
            def _():
                one_pass(j, k)
        return 0

    lax.fori_loop(1, 17, merge_body, 0)

    K = K_ref[...]
    gs_ref[...] = lax.bitcast_convert_type(K ^ ((K >> 31) & 0x7FFFFFFF),
                                           jnp.float32)
    iq_ref[...] = I_ref[...]


_tc_sort = pl.pallas_call(
    _sort_body,
    out_shape=(
        jax.ShapeDtypeStruct((RS, CS), jnp.float32),
        jax.ShapeDtypeStruct((RS, CS), jnp.int32),
    ),
    scratch_shapes=[
        pltpu.VMEM((RS, CS), jnp.int32),
        pltpu.VMEM((RS, CS), jnp.int32),
    ],
)


@functools.cache
def _sc_gather_kernel():
    mesh = plsc.VectorSubcoreMesh(core_axis_name="c", subcore_axis_name="s",
                                  num_cores=2, num_subcores=16)

    @functools.partial(
        pl.kernel,
        out_type=(
            jax.ShapeDtypeStruct((N, D), jnp.float32),
            jax.ShapeDtypeStruct((NH, L), jnp.int32),
            jax.ShapeDtypeStruct((NH,), jnp.float32),
        ),
        mesh=mesh,
        scratch_types=[
            pltpu.VMEM((S_PW,), jnp.int32),
            pltpu.VMEM((Q_PW,), jnp.int32),
            pltpu.VMEM((Q_PW,), jnp.int32),
            pltpu.VMEM((Q_PW,), jnp.int32),
            pltpu.VMEM((16,), jnp.int32),
            pltpu.VMEM((S_CH, D), jnp.float32),
            pltpu.VMEM((S_CH, D), jnp.float32),
            pltpu.VMEM((Q_CH, L), jnp.int32),
            pltpu.VMEM((Q_CH, L), jnp.int32),
            pltpu.VMEM((Q_PW,), jnp.float32),
            pltpu.SemaphoreType.DMA,
            pltpu.SemaphoreType.DMA,
            pltpu.SemaphoreType.DMA,
            pltpu.SemaphoreType.DMA,
        ],
        compiler_params=pltpu.CompilerParams(use_tc_tiling_on_sc=False,
                                             needs_layout_passes=False),
    )
    def sc_gather(states_hbm, src_hbm, iqp_hbm, iqq_hbm, lg_hbm, pos_hbm,
                  outst_hbm, outsamp_hbm, outlg_hbm,
                  sidx_v, iqq_v, ridx_v, hv_v, pos_v,
                  st0, st1, sb0, sb1, lg_v,
                  g0, g1, o0, o1):
        wid = lax.axis_index("s") * 2 + lax.axis_index("c")
        sbase = wid * S_PW
        qbase = wid * Q_PW

        pltpu.sync_copy(iqp_hbm.at[pl.ds(sbase, S_PW)], sidx_v)
        pltpu.sync_copy(iqq_hbm.at[pl.ds(qbase, Q_PW)], iqq_v)
        pltpu.sync_copy(pos_hbm, pos_v)
        colv = pos_v[...]

        def prep_s(i, _):
            o = pl.multiple_of(i * 16, 16)
            sidx_v[pl.ds(o, 16)] = sidx_v[pl.ds(o, 16)] >> 2
            return 0
        lax.fori_loop(0, S_PW // 16, prep_s, 0)

        def prep_q(i, _):
            o = pl.multiple_of(i * 16, 16)
            v = iqq_v[pl.ds(o, 16)]
            ridx_v[pl.ds(o, 16)] = v & -4
            hv_v[pl.ds(o, 16)] = v & 3
            return 0
        lax.fori_loop(0, Q_PW // 16, prep_q, 0)

        stb = (st0, st1)
        gsem = (g0, g1)
        osem = (o0, o1)

        def st_gather(c, b):
            return pltpu.make_async_copy(
                states_hbm.at[sidx_v.at[pl.ds(c * S_CH, S_CH)]], stb[b],
                gsem[b])

        def st_out(c, b):
            return pltpu.make_async_copy(
                stb[b], outst_hbm.at[pl.ds(sbase + c * S_CH, S_CH)], osem[b])

        nst = S_PW // S_CH
        st_gather(0, 0).start()
        for c in range(nst):
            b = c & 1
            if c + 1 < nst:
                if c >= 1:
                    st_out(c - 1, 1 - b).wait()
                st_gather(c + 1, 1 - b).start()
            st_gather(c, b).wait()
            st_out(c, b).start()
        st_out(nst - 2, nst & 1).wait()
        st_out(nst - 1, (nst - 1) & 1).wait()

        sbf = (sb0, sb1)

        def sp_gather(c, b):
            return pltpu.make_async_copy(
                src_hbm.at[ridx_v.at[pl.ds(c * Q_CH, Q_CH)]], sbf[b],
                gsem[b])

        def sp_out(c, b):
            return pltpu.make_async_copy(
                sbf[b], outsamp_hbm.at[pl.ds(qbase + c * Q_CH, Q_CH)],
                osem[b])

        nsp = Q_PW // Q_CH
        sp_gather(0, 0).start()
        for c in range(nsp):
            b = c & 1
            if c + 1 < nsp:
                if c >= 1:
                    sp_out(c - 1, 1 - b).wait()
                sp_gather(c + 1, 1 - b).start()
            sp_gather(c, b).wait()
            for t in range(Q_CH // 16):
                rvec = lax.iota(jnp.int32, 16) + t * 16
                hvec = hv_v[pl.ds(c * Q_CH + t * 16, 16)]
                plsc.store_scatter(sbf[b], (rvec, colv), hvec)
            sp_out(c, b).start()
        sp_out(nsp - 2, nsp & 1).wait()
        sp_out(nsp - 1, (nsp - 1) & 1).wait()

        nlg = Q_PW // Q_CH
        for t in range(nlg):
            pltpu.make_async_copy(
                lg_hbm.at[iqq_v.at[pl.ds(t * Q_CH, Q_CH)]],
                lg_v.at[pl.ds(t * Q_CH, Q_CH)], g0).start()
        for t in range(nlg):
            pltpu.make_async_copy(
                lg_hbm.at[iqq_v.at[pl.ds(t * Q_CH, Q_CH)]],
                lg_v.at[pl.ds(t * Q_CH, Q_CH)], g0).wait()
        pltpu.sync_copy(lg_v, outlg_hbm.at[pl.ds(qbase, Q_PW)])

    return sc_gather


def kernel(sample, logits, gumbel, net_logits, gumbel_noise, states, position):
    logitnew = logits[:, 0:1] + net_logits
    graw = logitnew + gumbel_noise
    Z = jnp.nanmax(graw, axis=1, keepdims=True)
    gnew = jnp.nan_to_num(
        -jnp.log(jnp.exp(-gumbel[:, 0:1]) - jnp.exp(-Z) + jnp.exp(-graw)),
        nan=-jnp.inf,
    )

    gs, iq = _tc_sort(gnew.reshape(RS, CS))

    out_gumbel = gs.reshape(H, N).T
    iqp = iq.reshape(-1)
    iqq = iq.reshape(H, N).T.reshape(-1)

    posv = jnp.full((16,), position, dtype=jnp.int32)
    out_states, out_samp, out_lg = _sc_gather_kernel()(
        states, sample.reshape(NH, L), iqp, iqq, logitnew.reshape(-1), posv)

    return (out_samp.reshape(N, H, L), out_lg.reshape(N, H), out_gumbel,
            out_states)


# speedup vs baseline: 4.5619x; 1.2249x over previous
import functools

import jax
import jax.numpy as jnp
from jax import lax
from jax.experimental import pallas as pl
from jax.experimental.pallas import tpu as pltpu
from jax.experimental.pallas import tpu_sc as plsc

N, H, L, D = 16384, 4, 200, 256
NH = N * H
RS, CS = 512, 128
NW = 32

S_PW, S_CH = N // NW, 64
Q_PW, Q_CH = NH // NW, 128
NB = 8


def _sort_body(g_ref, bt_ref, gs_ref, iq_ref, base_ref, K_ref, I_ref):
    for i in range(16):
        blk = bt_ref[:, pl.ds(i * 1024, 1024)]
        base_ref[pl.ds(i * 1024, 1024), :] = blk.T

    rowi = lax.broadcasted_iota(jnp.int32, (RS, CS), 0)
    coli = lax.broadcasted_iota(jnp.int32, (RS, CS), 1)
    fio = rowi * CS + coli

    u = lax.bitcast_convert_type(g_ref[...], jnp.int32)
    K_ref[...] = u ^ ((u >> 31) & 0x7FFFFFFF)
    I_ref[...] = fio

    def one_pass(j, k):
        K = K_ref[...]
        I = I_ref[...]
        d = 1 << j
        if j < 7:
            axis, s, size = 1, d, CS
            bsel = (coli & d) != 0
        else:
            axis, s, size = 0, d >> 7, RS
            bsel = (rowi & (d >> 7)) != 0

        def pt(x):
            return jnp.where(bsel, pltpu.roll(x, s, axis),
                             pltpu.roll(x, size - s, axis))

        Kp, Ip = pt(K), pt(I)
        Lt = (K > Kp) | ((K == Kp) & (I < Ip))
        dirb = ((fio >> k) & 1) != 0
        keep = Lt ^ dirb ^ bsel
        K_ref[...] = jnp.where(keep, K, Kp)
        I_ref[...] = jnp.where(keep, I, Ip)

    def merge_body(k, _):
        for j in range(15, -1, -1):
            @pl.when(j < k)
            def _():
                one_pass(j, k)
        return 0

    lax.fori_loop(1, 17, merge_body, 0)

    K = K_ref[...]
    gs_ref[...] = lax.bitcast_convert_type(K ^ ((K >> 31) & 0x7FFFFFFF),
                                           jnp.float32)
    iq_ref[...] = I_ref[...]


_tc_sort = pl.pallas_call(
    _sort_body,
    out_shape=(
        jax.ShapeDtypeStruct((RS, CS), jnp.float32),
        jax.ShapeDtypeStruct((RS, CS), jnp.int32),
        jax.ShapeDtypeStruct((N, L), jnp.int32),
    ),
    scratch_shapes=[
        pltpu.VMEM((RS, CS), jnp.int32),
        pltpu.VMEM((RS, CS), jnp.int32),
    ],
)


@functools.cache
def _sc_gather_kernel():
    mesh = plsc.VectorSubcoreMesh(core_axis_name="c", subcore_axis_name="s",
                                  num_cores=2, num_subcores=16)

    @functools.partial(
        pl.kernel,
        out_type=(
            jax.ShapeDtypeStruct((N, D), jnp.float32),
            jax.ShapeDtypeStruct((N, H, L), jnp.int32),
            jax.ShapeDtypeStruct((NH,), jnp.float32),
        ),
        mesh=mesh,
        scratch_types=[
            pltpu.VMEM((S_PW,), jnp.int32),
            pltpu.VMEM((Q_PW,), jnp.int32),
            pltpu.VMEM((Q_PW,), jnp.int32),
            pltpu.VMEM((Q_PW,), jnp.int32),
            pltpu.VMEM((16,), jnp.int32),
            pltpu.VMEM((S_CH, D), jnp.float32),
            pltpu.VMEM((S_CH, D), jnp.float32),
            pltpu.VMEM((Q_CH, L), jnp.int32),
            pltpu.VMEM((Q_CH, L), jnp.int32),
            pltpu.VMEM((Q_PW,), jnp.float32),
            pltpu.SemaphoreType.DMA,
            pltpu.SemaphoreType.DMA,
            pltpu.SemaphoreType.DMA,
            pltpu.SemaphoreType.DMA,
        ],
        compiler_params=pltpu.CompilerParams(use_tc_tiling_on_sc=False,
                                             needs_layout_passes=False),
    )
    def sc_gather(states_hbm, base_hbm, iqp_hbm, lg_hbm, pos_hbm,
                  outst_hbm, outsamp_hbm, outlg_hbm,
                  sidx_v, iq_v, ridx_v, hv_v, pos_v,
                  st0, st1, sb0, sb1, lg_v,
                  g0, g1, o0, o1):
        wid = lax.axis_index("s") * 2 + lax.axis_index("c")
        sbase = wid * S_PW
        pbase = wid * Q_PW
        hband = wid // NB
        nb0 = (wid % NB) * Q_PW

        pltpu.sync_copy(iqp_hbm.at[pl.ds(sbase, S_PW)], sidx_v)
        pltpu.sync_copy(iqp_hbm.at[pl.ds(pbase, Q_PW)], iq_v)
        pltpu.sync_copy(pos_hbm, pos_v)
        colv = pos_v[...]

        def prep_s(i, _):
            o = pl.multiple_of(i * 16, 16)
            sidx_v[pl.ds(o, 16)] = sidx_v[pl.ds(o, 16)] >> 2
            return 0
        lax.fori_loop(0, S_PW // 16, prep_s, 0)

        def prep_q(i, _):
            o = pl.multiple_of(i * 16, 16)
            v = iq_v[pl.ds(o, 16)]
            ridx_v[pl.ds(o, 16)] = v >> 2
            hv_v[pl.ds(o, 16)] = v & 3
            return 0
        lax.fori_loop(0, Q_PW // 16, prep_q, 0)

        stb = (st0, st1)
        gsem = (g0, g1)
        osem = (o0, o1)

        def st_gather(c, b):
            return pltpu.make_async_copy(
                states_hbm.at[sidx_v.at[pl.ds(c * S_CH, S_CH)]], stb[b],
                gsem[b])

        def st_out(c, b):
            return pltpu.make_async_copy(
                stb[b], outst_hbm.at[pl.ds(sbase + c * S_CH, S_CH)], osem[b])

        nst = S_PW // S_CH
        st_gather(0, 0).start()
        for c in range(nst):
            b = c & 1
            if c + 1 < nst:
                if c >= 1:
                    st_out(c - 1, 1 - b).wait()
                st_gather(c + 1, 1 - b).start()
            st_gather(c, b).wait()
            st_out(c, b).start()
        st_out(nst - 2, nst & 1).wait()
        st_out(nst - 1, (nst - 1) & 1).wait()

        sbf = (sb0, sb1)

        def sp_gather(c, b):
            return pltpu.make_async_copy(
                base_hbm.at[ridx_v.at[pl.ds(c * Q_CH, Q_CH)]], sbf[b],
                gsem[b])

        def sp_out(c, b):
            return pltpu.make_async_copy(
                sbf[b], outsamp_hbm.at[pl.ds(nb0 + c * Q_CH, Q_CH), hband],
                osem[b])

        nsp = Q_PW // Q_CH
        sp_gather(0, 0).start()
        for c in range(nsp):
            b = c & 1
            if c + 1 < nsp:
                if c >= 1:
                    sp_out(c - 1, 1 - b).wait()
                sp_gather(c + 1, 1 - b).start()
            sp_gather(c, b).wait()
            for t in range(Q_CH // 16):
                rvec = lax.iota(jnp.int32, 16) + t * 16
                hvec = hv_v[pl.ds(c * Q_CH + t * 16, 16)]
                plsc.store_scatter(sbf[b], (rvec, colv), hvec)
            sp_out(c, b).start()
        sp_out(nsp - 2, nsp & 1).wait()
        sp_out(nsp - 1, (nsp - 1) & 1).wait()

        nlg = Q_PW // Q_CH
        for t in range(nlg):
            pltpu.make_async_copy(
                lg_hbm.at[iq_v.at[pl.ds(t * Q_CH, Q_CH)]],
                lg_v.at[pl.ds(t * Q_CH, Q_CH)], g0).start()
        for t in range(nlg):
            pltpu.make_async_copy(
                lg_hbm.at[iq_v.at[pl.ds(t * Q_CH, Q_CH)]],
                lg_v.at[pl.ds(t * Q_CH, Q_CH)], g0).wait()
        pltpu.sync_copy(lg_v, outlg_hbm.at[pl.ds(pbase, Q_PW)])

    return sc_gather


def kernel(sample, logits, gumbel, net_logits, gumbel_noise, states, position):
    logitnew = logits[:, 0:1] + net_logits
    graw = logitnew + gumbel_noise
    Z = jnp.nanmax(graw, axis=1, keepdims=True)
    gnew = jnp.nan_to_num(
        -jnp.log(jnp.exp(-gumbel[:, 0:1]) - jnp.exp(-Z) + jnp.exp(-graw)),
        nan=-jnp.inf,
    )

    base_t = jnp.transpose(sample, (1, 2, 0))[0]
    gs, iq, base = _tc_sort(gnew.reshape(RS, CS), base_t)

    out_gumbel = gs.reshape(H, N).T
    iqp = iq.reshape(-1)

    posv = jnp.full((16,), position, dtype=jnp.int32)
    out_states, out_samp, out_lg = _sc_gather_kernel()(
        states, base, iqp, logitnew.reshape(-1), posv)

    out_logits = out_lg.reshape(H, N).T
    return (out_samp, out_logits, out_gumbel, out_states)


# speedup vs baseline: 5.4712x; 1.1993x over previous
import functools

import jax
import jax.numpy as jnp
from jax import lax
from jax.experimental import pallas as pl
from jax.experimental.pallas import tpu as pltpu
from jax.experimental.pallas import tpu_sc as plsc

N, H, L, D = 16384, 4, 200, 256
NH = N * H
RS, CS = 512, 128
NW = 32

S_PW, S_CH = N // NW, 64
Q_PW, Q_CH = NH // NW, 128
NB = 8


def _sort_body(g_ref, bt_ref, gs_ref, iq_ref, base_ref, K_ref, I_ref):
    for i in range(16):
        blk = bt_ref[:, pl.ds(i * 1024, 1024)]
        base_ref[pl.ds(i * 1024, 1024), :] = blk.T

    rowi = lax.broadcasted_iota(jnp.int32, (RS, CS), 0)
    coli = lax.broadcasted_iota(jnp.int32, (RS, CS), 1)
    fio = rowi * CS + coli

    u = lax.bitcast_convert_type(g_ref[...], jnp.int32)
    K_ref[...] = u ^ ((u >> 31) & 0x7FFFFFFF)
    I_ref[...] = ((fio & (N - 1)) << 2) | (fio >> 14)

    def one_pass(j, k):
        K = K_ref[...]
        I = I_ref[...]
        d = 1 << j
        if j < 7:
            axis, s, size = 1, d, CS
            bsel = (coli & d) != 0
        else:
            axis, s, size = 0, d >> 7, RS
            bsel = (rowi & (d >> 7)) != 0

        def pt(x):
            return jnp.where(bsel, pltpu.roll(x, s, axis),
                             pltpu.roll(x, size - s, axis))

        Kp, Ip = pt(K), pt(I)
        Lt = (K > Kp) | ((K == Kp) & (I < Ip))
        dirb = ((fio >> k) & 1) != 0
        keep = Lt ^ dirb ^ bsel
        K_ref[...] = jnp.where(keep, K, Kp)
        I_ref[...] = jnp.where(keep, I, Ip)

    def merge_body(k, _):
        for j in range(15, -1, -1):
            @pl.when(j < k)
            def _():
                one_pass(j, k)
        return 0

    lax.fori_loop(1, 17, merge_body, 0)

    K = K_ref[...]
    gs_ref[...] = lax.bitcast_convert_type(K ^ ((K >> 31) & 0x7FFFFFFF),
                                           jnp.float32)
    iq_ref[...] = I_ref[...]


_tc_sort = pl.pallas_call(
    _sort_body,
    out_shape=(
        jax.ShapeDtypeStruct((RS, CS), jnp.float32),
        jax.ShapeDtypeStruct((RS, CS), jnp.int32),
        jax.ShapeDtypeStruct((N, L), jnp.int32),
    ),
    scratch_shapes=[
        pltpu.VMEM((RS, CS), jnp.int32),
        pltpu.VMEM((RS, CS), jnp.int32),
    ],
)


@functools.cache
def _sc_kernels():
    mesh = plsc.VectorSubcoreMesh(core_axis_name="c", subcore_axis_name="s",
                                  num_cores=2, num_subcores=16)

    @functools.partial(
        pl.kernel,
        out_type=jax.ShapeDtypeStruct((N, D), jnp.float32),
        mesh=mesh,
        scratch_types=[
            pltpu.VMEM((S_PW,), jnp.int32),
            pltpu.VMEM((S_CH, D), jnp.float32),
            pltpu.VMEM((S_CH, D), jnp.float32),
            pltpu.SemaphoreType.DMA,
            pltpu.SemaphoreType.DMA,
            pltpu.SemaphoreType.DMA,
            pltpu.SemaphoreType.DMA,
        ],
    )
    def sc_states(states_hbm, iqp_hbm, out_hbm, sidx_v, st0, st1,
                  g0, g1, o0, o1):
        wid = lax.axis_index("s") * 2 + lax.axis_index("c")
        sbase = wid * S_PW
        pltpu.sync_copy(iqp_hbm.at[pl.ds(sbase, S_PW)], sidx_v)

        def prep_s(i, _):
            o = pl.multiple_of(i * 16, 16)
            sidx_v[pl.ds(o, 16)] = sidx_v[pl.ds(o, 16)] >> 2
            return 0
        lax.fori_loop(0, S_PW // 16, prep_s, 0)

        stb = (st0, st1)
        gsem = (g0, g1)
        osem = (o0, o1)

        def st_gather(c, b):
            return pltpu.make_async_copy(
                states_hbm.at[sidx_v.at[pl.ds(c * S_CH, S_CH)]], stb[b],
                gsem[b])

        def st_out(c, b):
            return pltpu.make_async_copy(
                stb[b], out_hbm.at[pl.ds(sbase + c * S_CH, S_CH)], osem[b])

        nst = S_PW // S_CH
        st_gather(0, 0).start()
        for c in range(nst):
            b = c & 1
            if c + 1 < nst:
                if c >= 1:
                    st_out(c - 1, 1 - b).wait()
                st_gather(c + 1, 1 - b).start()
            st_gather(c, b).wait()
            st_out(c, b).start()
        st_out(nst - 2, nst & 1).wait()
        st_out(nst - 1, (nst - 1) & 1).wait()

    @functools.partial(
        pl.kernel,
        out_type=(
            jax.ShapeDtypeStruct((H, N, L), jnp.int32),
            jax.ShapeDtypeStruct((NH,), jnp.float32),
        ),
        mesh=mesh,
        scratch_types=[
            pltpu.VMEM((Q_PW,), jnp.int32),
            pltpu.VMEM((Q_PW,), jnp.int32),
            pltpu.VMEM((Q_PW,), jnp.int32),
            pltpu.VMEM((16,), jnp.int32),
            pltpu.VMEM((Q_CH, L), jnp.int32),
            pltpu.VMEM((Q_CH, L), jnp.int32),
            pltpu.VMEM((Q_PW,), jnp.float32),
            pltpu.SemaphoreType.DMA,
            pltpu.SemaphoreType.DMA,
            pltpu.SemaphoreType.DMA,
            pltpu.SemaphoreType.DMA,
        ],
        compiler_params=pltpu.CompilerParams(use_tc_tiling_on_sc=False,
                                             needs_layout_passes=False),
    )
    def sc_sample(base_hbm, iqp_hbm, lg_hbm, pos_hbm,
                  outsamp_hbm, outlg_hbm,
                  iq_v, ridx_v, hv_v, pos_v, sb0, sb1, lg_v,
                  g0, g1, o0, o1):
        wid = lax.axis_index("s") * 2 + lax.axis_index("c")
        pbase = wid * Q_PW
        hband = wid // NB
        nb0 = (wid % NB) * Q_PW

        pltpu.sync_copy(iqp_hbm.at[pl.ds(pbase, Q_PW)], iq_v)
        pltpu.sync_copy(pos_hbm, pos_v)
        colv = pos_v[...]

        def prep_q(i, _):
            o = pl.multiple_of(i * 16, 16)
            v = iq_v[pl.ds(o, 16)]
            ridx_v[pl.ds(o, 16)] = v >> 2
            hv_v[pl.ds(o, 16)] = v & 3
            iq_v[pl.ds(o, 16)] = ((v & 3) << 14) | (v >> 2)
            return 0
        lax.fori_loop(0, Q_PW // 16, prep_q, 0)

        gsem = (g0, g1)
        osem = (o0, o1)
        sbf = (sb0, sb1)

        def sp_gather(c, b):
            return pltpu.make_async_copy(
                base_hbm.at[ridx_v.at[pl.ds(c * Q_CH, Q_CH)]], sbf[b],
                gsem[b])

        def sp_out(c, b):
            return pltpu.make_async_copy(
                sbf[b], outsamp_hbm.at[hband, pl.ds(nb0 + c * Q_CH, Q_CH)],
                osem[b])

        nsp = Q_PW // Q_CH
        sp_gather(0, 0).start()
        for c in range(nsp):
            b = c & 1
            if c + 1 < nsp:
                if c >= 1:
                    sp_out(c - 1, 1 - b).wait()
                sp_gather(c + 1, 1 - b).start()
            sp_gather(c, b).wait()
            for t in range(Q_CH // 16):
                rvec = lax.iota(jnp.int32, 16) + t * 16
                hvec = hv_v[pl.ds(c * Q_CH + t * 16, 16)]
                plsc.store_scatter(sbf[b], (rvec, colv), hvec)
            sp_out(c, b).start()
        sp_out(nsp - 2, nsp & 1).wait()
        sp_out(nsp - 1, (nsp - 1) & 1).wait()

        nlg = Q_PW // Q_CH
        for t in range(nlg):
            pltpu.make_async_copy(
                lg_hbm.at[iq_v.at[pl.ds(t * Q_CH, Q_CH)]],
                lg_v.at[pl.ds(t * Q_CH, Q_CH)], g0).start()
        for t in range(nlg):
            pltpu.make_async_copy(
                lg_hbm.at[iq_v.at[pl.ds(t * Q_CH, Q_CH)]],
                lg_v.at[pl.ds(t * Q_CH, Q_CH)], g0).wait()
        pltpu.sync_copy(lg_v, outlg_hbm.at[pl.ds(pbase, Q_PW)])

    return sc_states, sc_sample


def kernel(sample, logits, gumbel, net_logits, gumbel_noise, states, position):
    logits_t = jnp.transpose(logits)
    gumbel_t = jnp.transpose(gumbel)
    net_t = jnp.transpose(net_logits)
    noise_t = jnp.transpose(gumbel_noise)
    logitnew_t = logits_t[0:1, :] + net_t
    graw_t = logitnew_t + noise_t
    Z_t = jnp.nanmax(graw_t, axis=0, keepdims=True)
    gnew_t = jnp.nan_to_num(
        -jnp.log(jnp.exp(-gumbel_t[0:1, :]) - jnp.exp(-Z_t)
                 + jnp.exp(-graw_t)),
        nan=-jnp.inf,
    )

    base_t = jnp.transpose(sample, (1, 2, 0))[0]
    gs, iq, base = _tc_sort(gnew_t.reshape(RS, CS), base_t)

    out_gumbel = gs.reshape(H, N).T
    iqp = iq.reshape(-1)

    sc_states, sc_sample = _sc_kernels()
    out_states = sc_states(states, iqp)

    posv = jnp.full((16,), position, dtype=jnp.int32)
    out_samp, out_lg = sc_sample(base, iqp, logitnew_t.reshape(-1), posv)

    out_logits = out_lg.reshape(H, N).T
    return (out_samp.transpose(1, 0, 2), out_logits, out_gumbel, out_states)
